# Initial kernel scaffold; baseline (speedup 1.0000x reference)
import jax
import jax.numpy as jnp
from jax.experimental import pallas as pl


def kernel(x, edge_index, batch, c0_W1, c0_b1, c0_g1, c0_be1, c0_W2, c0_b2, bn0_g, bn0_b, c1_W1, c1_b1, c1_g1, c1_be1, c1_W2, c1_b2, bn1_g, bn1_b, l0_W, l0_b, l1_W, l1_b, l2_W, l2_b):
    raise NotImplementedError("write your pallas kernel here")


# speedup vs baseline: 5.8290x; 5.8290x over previous
import functools

import jax
import jax.numpy as jnp
import numpy as np
from jax import lax
from jax.experimental import pallas as pl
from jax.experimental.pallas import tpu as pltpu
from jax.experimental.pallas import tpu_sc as plsc

_NC = 2
_NS = 16
_B = 128
_G = 64


def _make_sc_aggr(n_nodes, fc, edges_per_core, edge_stride, src_core_stride):
    assert edges_per_core % _NS == 0
    ew = edges_per_core // _NS
    nfull = ew // _B
    rem = ew - nfull * _B
    nfull2 = (nfull // 2) * 2
    assert nfull2 >= 2
    nrow = ((n_nodes + 1 + _NS * 16 - 1) // (_NS * 16)) * (_NS * 16)
    rows_pt = nrow // _NS

    mesh = plsc.VectorSubcoreMesh(core_axis_name="c", subcore_axis_name="s")

    @functools.partial(
        pl.kernel,
        out_type=jax.ShapeDtypeStruct((_NC, nrow, fc), jnp.float32),
        mesh=mesh,
        scratch_types=[
            pltpu.VMEM((2, _B), jnp.int32),
            pltpu.VMEM((2, _B), jnp.int32),
            pltpu.VMEM((_B, fc), jnp.float32),
            pltpu.VMEM((_B, fc), jnp.float32),
            pltpu.VMEM((16, fc), jnp.float32),
            pltpu.VMEM_SHARED((nrow, fc), jnp.float32),
            pltpu.SemaphoreType.DMA,
            pltpu.SemaphoreType.DMA,
            pltpu.SemaphoreType.DMA,
            pltpu.SemaphoreType.DMA,
        ],
    )
    def aggr(src_hbm, dst_hbm, table_hbm, out_hbm,
             srcb, dstb, rows0, rows1, zb, accum, si0, si1, sg0, sg1):
        c = lax.axis_index("c")
        s = lax.axis_index("s")
        dbase = c * edge_stride + s * ew
        sbase = c * src_core_stride + s * ew
        rowsb = (rows0, rows1)
        gsem = (sg0, sg1)
        isem = (si0, si1)

        def idx_pair(jj, par):
            return (
                pltpu.make_async_copy(
                    src_hbm.at[pl.ds(sbase + jj * _B, _B)], srcb.at[par],
                    isem[par]),
                pltpu.make_async_copy(
                    dst_hbm.at[pl.ds(dbase + jj * _B, _B)], dstb.at[par],
                    isem[par]),
            )

        def start_idx(jj, par):
            a, b = idx_pair(jj, par)
            a.start()
            b.start()

        def wait_idx(jj, par):
            a, b = idx_pair(jj, par)
            a.wait()
            b.wait()

        def gather_desc(par):
            return pltpu.make_async_copy(
                table_hbm.at[srcb.at[par]], rowsb[par], gsem[par])

        def scatter(par):
            pltpu.sync_copy(rowsb[par], accum.at[dstb.at[par]], add=True)

        zv = jnp.zeros((16,), jnp.float32)
        for r in range(16):
            for k in range(fc // 16):
                zb[r, pl.ds(k * 16, 16)] = zv
        row0 = s * rows_pt

        @pl.loop(0, rows_pt // 16)
        def _zero(g):
            pltpu.sync_copy(zb, accum.at[pl.ds(row0 + g * 16, 16), :])

        start_idx(0, 0)
        start_idx(1, 1)
        plsc.subcore_barrier()
        wait_idx(0, 0)
        gather_desc(0).start()

        @pl.loop(0, nfull2, step=2)
        def _main(j):
            for par in (0, 1):
                jj = j + par
                gather_desc(par).wait()

                @pl.when(jj + 1 < nfull2)
                def _next_gather():
                    wait_idx(jj + 1, 1 - par)
                    gather_desc(1 - par).start()

                scatter(par)

                @pl.when(jj + 2 < nfull2)
                def _prefetch_idx():
                    start_idx(jj + 2, par)

        for lb in range(nfull2, nfull):
            start_idx(lb, 0)
            wait_idx(lb, 0)
            gather_desc(0).start()
            gather_desc(0).wait()
            scatter(0)
        if rem:
            zpad = jnp.zeros((16,), jnp.int32)
            dpad = jnp.full((16,), n_nodes, jnp.int32)
            for k in range(_B // 16):
                srcb[0, pl.ds(k * 16, 16)] = zpad
                dstb[0, pl.ds(k * 16, 16)] = dpad
            ra = pltpu.make_async_copy(
                src_hbm.at[pl.ds(sbase + nfull * _B, rem)],
                srcb.at[0, pl.ds(0, rem)], si0)
            rb = pltpu.make_async_copy(
                dst_hbm.at[pl.ds(dbase + nfull * _B, rem)],
                dstb.at[0, pl.ds(0, rem)], si0)
            ra.start()
            rb.start()
            ra.wait()
            rb.wait()
            gather_desc(0).start()
            gather_desc(0).wait()
            scatter(0)

        plsc.subcore_barrier()
        pltpu.sync_copy(accum.at[pl.ds(row0, rows_pt), :],
                        out_hbm.at[c, pl.ds(row0, rows_pt), :])

    return aggr, nrow


def _tc_conv0(n, f, h, rb):
    grid = n // rb

    def body(x_ref, a_ref, b_ref, w1_ref, b1_ref, w2_ref, b2_ref,
             h_ref, p0_ref, p1_ref):
        i = pl.program_id(0)
        x = x_ref[...]
        z = x + a_ref[0] + a_ref[1]
        t = jnp.maximum(
            jnp.dot(z, w1_ref[...], preferred_element_type=jnp.float32)
            + b1_ref[...], 0.0)
        hh = jnp.maximum(
            jnp.dot(t, w2_ref[...], preferred_element_type=jnp.float32)
            + b2_ref[...], 0.0)
        h_ref[0] = hh[:, :f]
        h_ref[1] = hh[:, f:]
        gidx = lax.broadcasted_iota(jnp.int32, (1, _G), 1)
        oh = (b_ref[...] == gidx).astype(jnp.float32)
        dn = (((0,), (0,)), ((), ()))
        p0c = lax.dot_general(oh, x, dn, preferred_element_type=jnp.float32)
        p1c = lax.dot_general(oh, hh, dn, preferred_element_type=jnp.float32)

        @pl.when(i == 0)
        def _init():
            p0_ref[...] = p0c
            p1_ref[...] = p1c

        @pl.when(i > 0)
        def _acc():
            p0_ref[...] += p0c
            p1_ref[...] += p1c

    return body, grid


def _tc_conv1(n, f, h, c_out, rb):
    grid = n // rb

    def body(h1_ref, a_ref, b_ref, w1lo_ref, w1hi_ref, b1_ref, w2_ref, b2_ref,
             p0_ref, p1_ref, l0_ref, l1_ref, l2_ref, bs_ref,
             out_ref, p2_acc):
        i = pl.program_id(0)
        zlo = h1_ref[0] + a_ref[0]
        zhi = h1_ref[1] + a_ref[1]
        t = jnp.maximum(
            jnp.dot(zlo, w1lo_ref[...], preferred_element_type=jnp.float32)
            + jnp.dot(zhi, w1hi_ref[...], preferred_element_type=jnp.float32)
            + b1_ref[...], 0.0)
        h2 = jnp.maximum(
            jnp.dot(t, w2_ref[...], preferred_element_type=jnp.float32)
            + b2_ref[...], 0.0)
        gidx = lax.broadcasted_iota(jnp.int32, (1, _G), 1)
        oh = (b_ref[...] == gidx).astype(jnp.float32)
        dn = (((0,), (0,)), ((), ()))
        p2c = lax.dot_general(oh, h2, dn, preferred_element_type=jnp.float32)

        @pl.when(i == 0)
        def _init():
            p2_acc[...] = p2c

        @pl.when(i > 0)
        def _acc():
            p2_acc[...] += p2c

        @pl.when(i == grid - 1)
        def _final():
            out_ref[...] = (
                jnp.dot(p0_ref[...], l0_ref[...],
                        preferred_element_type=jnp.float32)
                + jnp.dot(p1_ref[...], l1_ref[...],
                          preferred_element_type=jnp.float32)
                + jnp.dot(p2_acc[...], l2_ref[...],
                          preferred_element_type=jnp.float32)
                + bs_ref[...])

    return body, grid


def kernel(x, edge_index, batch,
           c0_W1, c0_b1, c0_g1, c0_be1, c0_W2, c0_b2, bn0_g, bn0_b,
           c1_W1, c1_b1, c1_g1, c1_be1, c1_W2, c1_b2, bn1_g, bn1_b,
           l0_W, l0_b, l1_W, l1_b, l2_W, l2_b):
    n, f = x.shape
    e = edge_index.shape[1]
    h = c0_W2.shape[0]
    c_out = l0_W.shape[1]
    rb = 2000

    src = edge_index[0]
    dst = edge_index[1]
    batch2d = batch.reshape(n, 1)

    inv = float(1.0 / np.sqrt(1.0 + 1e-5))
    s1a = c0_g1 * inv
    w1f0 = c0_W1 * s1a[None, :]
    b1f0 = (c0_b1 * s1a + c0_be1).reshape(1, h)
    s0 = bn0_g * inv
    w2f0 = c0_W2 * s0[None, :]
    b2f0 = (c0_b2 * s0 + bn0_b).reshape(1, h)
    s1b = c1_g1 * inv
    w1f1 = c1_W1 * s1b[None, :]
    b1f1 = (c1_b1 * s1b + c1_be1).reshape(1, h)
    s1 = bn1_g * inv
    w2f1 = c1_W2 * s1[None, :]
    b2f1 = (c1_b2 * s1 + bn1_b).reshape(1, h)
    bsum = (l0_b + l1_b + l2_b).reshape(1, c_out)

    sc0, nrow = _make_sc_aggr(n, f, e // 2, e // 2, e // 2)
    aggr0 = sc0(src, dst, x)

    body0, grid0 = _tc_conv0(n, f, h, rb)
    h1, pooled0, pooled1 = pl.pallas_call(
        body0,
        grid=(grid0,),
        in_specs=[
            pl.BlockSpec((rb, f), lambda i: (i, 0)),
            pl.BlockSpec((2, rb, f), lambda i: (0, i, 0)),
            pl.BlockSpec((rb, 1), lambda i: (i, 0)),
            pl.BlockSpec((f, h), lambda i: (0, 0)),
            pl.BlockSpec((1, h), lambda i: (0, 0)),
            pl.BlockSpec((h, h), lambda i: (0, 0)),
            pl.BlockSpec((1, h), lambda i: (0, 0)),
        ],
        out_specs=[
            pl.BlockSpec((2, rb, f), lambda i: (0, i, 0)),
            pl.BlockSpec((_G, f), lambda i: (0, 0)),
            pl.BlockSpec((_G, h), lambda i: (0, 0)),
        ],
        out_shape=[
            jax.ShapeDtypeStruct((2, n, f), jnp.float32),
            jax.ShapeDtypeStruct((_G, f), jnp.float32),
            jax.ShapeDtypeStruct((_G, h), jnp.float32),
        ],
    )(x, aggr0, batch2d, w1f0, b1f0, w2f0, b2f0)

    h1flat = h1.reshape(2 * n, f)
    src_cat = jnp.concatenate([src, src + n])
    sc1, nrow1 = _make_sc_aggr(n, f, e, 0, e)
    aggr1 = sc1(src_cat, dst, h1flat)

    body1, grid1 = _tc_conv1(n, f, h, c_out, rb)
    out = pl.pallas_call(
        body1,
        grid=(grid1,),
        in_specs=[
            pl.BlockSpec((2, rb, f), lambda i: (0, i, 0)),
            pl.BlockSpec((2, rb, f), lambda i: (0, i, 0)),
            pl.BlockSpec((rb, 1), lambda i: (i, 0)),
            pl.BlockSpec((f, h), lambda i: (0, 0)),
            pl.BlockSpec((f, h), lambda i: (0, 0)),
            pl.BlockSpec((1, h), lambda i: (0, 0)),
            pl.BlockSpec((h, h), lambda i: (0, 0)),
            pl.BlockSpec((1, h), lambda i: (0, 0)),
            pl.BlockSpec((_G, f), lambda i: (0, 0)),
            pl.BlockSpec((_G, h), lambda i: (0, 0)),
            pl.BlockSpec((f, c_out), lambda i: (0, 0)),
            pl.BlockSpec((h, c_out), lambda i: (0, 0)),
            pl.BlockSpec((h, c_out), lambda i: (0, 0)),
            pl.BlockSpec((1, c_out), lambda i: (0, 0)),
        ],
        out_specs=pl.BlockSpec((_G, c_out), lambda i: (0, 0)),
        out_shape=jax.ShapeDtypeStruct((_G, c_out), jnp.float32),
        scratch_shapes=[pltpu.VMEM((_G, h), jnp.float32)],
    )(h1, aggr1, batch2d, w1f1[:f], w1f1[f:], b1f1, w2f1, b2f1,
      pooled0, pooled1, l0_W, l1_W, l2_W, bsum)

    return out


# speedup vs baseline: 7.9384x; 1.3619x over previous
import functools

import jax
import jax.numpy as jnp
import numpy as np
from jax import lax
from jax.experimental import pallas as pl
from jax.experimental.pallas import tpu as pltpu
from jax.experimental.pallas import tpu_sc as plsc

_NC = 2
_NS = 16
_B = 64
_NSL = 4
_G = 64


def _make_sc_aggr(n_nodes, fc, edges_per_core, edge_stride, src_core_stride):
    assert edges_per_core % _NS == 0
    ew = edges_per_core // _NS
    nfull = ew // _B
    rem = ew - nfull * _B
    n4 = (nfull // _NSL) * _NSL
    assert n4 >= _NSL
    nrow = ((n_nodes + 1 + _NS * 16 - 1) // (_NS * 16)) * (_NS * 16)
    rows_pt = nrow // _NS

    mesh = plsc.VectorSubcoreMesh(core_axis_name="c", subcore_axis_name="s")

    @functools.partial(
        pl.kernel,
        out_type=jax.ShapeDtypeStruct((_NC, nrow, fc), jnp.float32),
        mesh=mesh,
        scratch_types=[
            pltpu.VMEM((_NSL, _B), jnp.int32),
            pltpu.VMEM((_NSL, _B), jnp.int32),
            pltpu.VMEM((_NSL, _B, fc), jnp.float32),
            pltpu.VMEM((16, fc), jnp.float32),
            pltpu.VMEM_SHARED((nrow, fc), jnp.float32),
            [pltpu.SemaphoreType.DMA] * _NSL,
            [pltpu.SemaphoreType.DMA] * _NSL,
            [pltpu.SemaphoreType.DMA] * _NSL,
        ],
    )
    def aggr(src_hbm, dst_hbm, table_hbm, out_hbm,
             srcb, dstb, rows, zb, accum, isem, gsem, ssem):
        c = lax.axis_index("c")
        s = lax.axis_index("s")
        dbase = c * edge_stride + s * ew
        sbase = c * src_core_stride + s * ew

        def idx_pair(jj, sl):
            return (
                pltpu.make_async_copy(
                    src_hbm.at[pl.ds(sbase + jj * _B, _B)], srcb.at[sl],
                    isem[sl]),
                pltpu.make_async_copy(
                    dst_hbm.at[pl.ds(dbase + jj * _B, _B)], dstb.at[sl],
                    isem[sl]),
            )

        def start_idx(jj, sl):
            a, b = idx_pair(jj, sl)
            a.start()
            b.start()

        def wait_idx(jj, sl):
            a, b = idx_pair(jj, sl)
            a.wait()
            b.wait()

        def gather_desc(sl):
            return pltpu.make_async_copy(
                table_hbm.at[srcb.at[sl]], rows.at[sl], gsem[sl])

        def scatter_start(sl):
            pltpu.async_copy(rows.at[sl], accum.at[dstb.at[sl]], ssem[sl],
                             add=True)

        def scatter_wait(sl):
            pltpu.make_async_copy(rows.at[sl], accum.at[dstb.at[sl]],
                                  ssem[sl]).wait()

        zv = jnp.zeros((16,), jnp.float32)
        for r in range(16):
            for k in range(fc // 16):
                zb[r, pl.ds(k * 16, 16)] = zv
        row0 = s * rows_pt

        @pl.loop(0, rows_pt // 16)
        def _zero(g):
            pltpu.sync_copy(zb, accum.at[pl.ds(row0 + g * 16, 16), :])

        for sl in range(_NSL):
            start_idx(sl, sl)
        plsc.subcore_barrier()
        wait_idx(0, 0)
        gather_desc(0).start()
        wait_idx(1, 1)
        gather_desc(1).start()

        @pl.loop(0, n4, step=_NSL)
        def _main(j):
            for sl in range(_NSL):
                jj = j + sl
                sl2 = (sl + 2) % _NSL
                gather_desc(sl).wait()

                @pl.when(jj + _NSL < n4)
                def _pf_idx():
                    start_idx(jj + _NSL, sl)

                @pl.when(jnp.logical_and(jj >= 2, jj + 2 < n4))
                def _free_buf():
                    scatter_wait(sl2)

                @pl.when(jj + 2 < n4)
                def _next_gather():
                    wait_idx(jj + 2, sl2)
                    gather_desc(sl2).start()

                scatter_start(sl)

        for sl in range(min(_NSL, n4)):
            scatter_wait(sl)

        for lb in range(n4, nfull):
            start_idx(lb, 0)
            wait_idx(lb, 0)
            gather_desc(0).start()
            gather_desc(0).wait()
            scatter_start(0)
            scatter_wait(0)
        if rem:
            zpad = jnp.zeros((16,), jnp.int32)
            dpad = jnp.full((16,), n_nodes, jnp.int32)
            for k in range(_B // 16):
                srcb[0, pl.ds(k * 16, 16)] = zpad
                dstb[0, pl.ds(k * 16, 16)] = dpad
            ra = pltpu.make_async_copy(
                src_hbm.at[pl.ds(sbase + nfull * _B, rem)],
                srcb.at[0, pl.ds(0, rem)], isem[0])
            rb = pltpu.make_async_copy(
                dst_hbm.at[pl.ds(dbase + nfull * _B, rem)],
                dstb.at[0, pl.ds(0, rem)], isem[0])
            ra.start()
            rb.start()
            ra.wait()
            rb.wait()
            gather_desc(0).start()
            gather_desc(0).wait()
            scatter_start(0)
            scatter_wait(0)

        plsc.subcore_barrier()
        pltpu.sync_copy(accum.at[pl.ds(row0, rows_pt), :],
                        out_hbm.at[c, pl.ds(row0, rows_pt), :])

    return aggr, nrow


def _tc_conv0(n, f, h, rb):
    grid = n // rb

    def body(x_ref, a_ref, b_ref, w1_ref, b1_ref, w2_ref, b2_ref,
             h_ref, p0_ref, p1_ref):
        i = pl.program_id(0)
        x = x_ref[...]
        z = x + a_ref[0] + a_ref[1]
        t = jnp.maximum(
            jnp.dot(z, w1_ref[...], preferred_element_type=jnp.float32)
            + b1_ref[...], 0.0)
        hh = jnp.maximum(
            jnp.dot(t, w2_ref[...], preferred_element_type=jnp.float32)
            + b2_ref[...], 0.0)
        h_ref[0] = hh[:, :f]
        h_ref[1] = hh[:, f:]
        gidx = lax.broadcasted_iota(jnp.int32, (1, _G), 1)
        oh = (b_ref[...] == gidx).astype(jnp.float32)
        dn = (((0,), (0,)), ((), ()))
        p0c = lax.dot_general(oh, x, dn, preferred_element_type=jnp.float32)
        p1c = lax.dot_general(oh, hh, dn, preferred_element_type=jnp.float32)

        @pl.when(i == 0)
        def _init():
            p0_ref[...] = p0c
            p1_ref[...] = p1c

        @pl.when(i > 0)
        def _acc():
            p0_ref[...] += p0c
            p1_ref[...] += p1c

    return body, grid


def _tc_conv1(n, f, h, c_out, rb):
    grid = n // rb

    def body(h1_ref, a_ref, b_ref, w1lo_ref, w1hi_ref, b1_ref, w2_ref, b2_ref,
             p0_ref, p1_ref, l0_ref, l1_ref, l2_ref, bs_ref,
             out_ref, p2_acc):
        i = pl.program_id(0)
        zlo = h1_ref[0] + a_ref[0]
        zhi = h1_ref[1] + a_ref[1]
        t = jnp.maximum(
            jnp.dot(zlo, w1lo_ref[...], preferred_element_type=jnp.float32)
            + jnp.dot(zhi, w1hi_ref[...], preferred_element_type=jnp.float32)
            + b1_ref[...], 0.0)
        h2 = jnp.maximum(
            jnp.dot(t, w2_ref[...], preferred_element_type=jnp.float32)
            + b2_ref[...], 0.0)
        gidx = lax.broadcasted_iota(jnp.int32, (1, _G), 1)
        oh = (b_ref[...] == gidx).astype(jnp.float32)
        dn = (((0,), (0,)), ((), ()))
        p2c = lax.dot_general(oh, h2, dn, preferred_element_type=jnp.float32)

        @pl.when(i == 0)
        def _init():
            p2_acc[...] = p2c

        @pl.when(i > 0)
        def _acc():
            p2_acc[...] += p2c

        @pl.when(i == grid - 1)
        def _final():
            out_ref[...] = (
                jnp.dot(p0_ref[...], l0_ref[...],
                        preferred_element_type=jnp.float32)
                + jnp.dot(p1_ref[...], l1_ref[...],
                          preferred_element_type=jnp.float32)
                + jnp.dot(p2_acc[...], l2_ref[...],
                          preferred_element_type=jnp.float32)
                + bs_ref[...])

    return body, grid


def kernel(x, edge_index, batch,
           c0_W1, c0_b1, c0_g1, c0_be1, c0_W2, c0_b2, bn0_g, bn0_b,
           c1_W1, c1_b1, c1_g1, c1_be1, c1_W2, c1_b2, bn1_g, bn1_b,
           l0_W, l0_b, l1_W, l1_b, l2_W, l2_b):
    n, f = x.shape
    e = edge_index.shape[1]
    h = c0_W2.shape[0]
    c_out = l0_W.shape[1]
    rb = 2000

    src = edge_index[0]
    dst = edge_index[1]
    batch2d = batch.reshape(n, 1)

    inv = float(1.0 / np.sqrt(1.0 + 1e-5))
    s1a = c0_g1 * inv
    w1f0 = c0_W1 * s1a[None, :]
    b1f0 = (c0_b1 * s1a + c0_be1).reshape(1, h)
    s0 = bn0_g * inv
    w2f0 = c0_W2 * s0[None, :]
    b2f0 = (c0_b2 * s0 + bn0_b).reshape(1, h)
    s1b = c1_g1 * inv
    w1f1 = c1_W1 * s1b[None, :]
    b1f1 = (c1_b1 * s1b + c1_be1).reshape(1, h)
    s1 = bn1_g * inv
    w2f1 = c1_W2 * s1[None, :]
    b2f1 = (c1_b2 * s1 + bn1_b).reshape(1, h)
    bsum = (l0_b + l1_b + l2_b).reshape(1, c_out)

    sc0, nrow = _make_sc_aggr(n, f, e // 2, e // 2, e // 2)
    aggr0 = sc0(src, dst, x)

    body0, grid0 = _tc_conv0(n, f, h, rb)
    h1, pooled0, pooled1 = pl.pallas_call(
        body0,
        grid=(grid0,),
        in_specs=[
            pl.BlockSpec((rb, f), lambda i: (i, 0)),
            pl.BlockSpec((2, rb, f), lambda i: (0, i, 0)),
            pl.BlockSpec((rb, 1), lambda i: (i, 0)),
            pl.BlockSpec((f, h), lambda i: (0, 0)),
            pl.BlockSpec((1, h), lambda i: (0, 0)),
            pl.BlockSpec((h, h), lambda i: (0, 0)),
            pl.BlockSpec((1, h), lambda i: (0, 0)),
        ],
        out_specs=[
            pl.BlockSpec((2, rb, f), lambda i: (0, i, 0)),
            pl.BlockSpec((_G, f), lambda i: (0, 0)),
            pl.BlockSpec((_G, h), lambda i: (0, 0)),
        ],
        out_shape=[
            jax.ShapeDtypeStruct((2, n, f), jnp.float32),
            jax.ShapeDtypeStruct((_G, f), jnp.float32),
            jax.ShapeDtypeStruct((_G, h), jnp.float32),
        ],
    )(x, aggr0, batch2d, w1f0, b1f0, w2f0, b2f0)

    h1flat = h1.reshape(2 * n, f)
    src_cat = jnp.concatenate([src, src + n])
    sc1, nrow1 = _make_sc_aggr(n, f, e, 0, e)
    aggr1 = sc1(src_cat, dst, h1flat)

    body1, grid1 = _tc_conv1(n, f, h, c_out, rb)
    out = pl.pallas_call(
        body1,
        grid=(grid1,),
        in_specs=[
            pl.BlockSpec((2, rb, f), lambda i: (0, i, 0)),
            pl.BlockSpec((2, rb, f), lambda i: (0, i, 0)),
            pl.BlockSpec((rb, 1), lambda i: (i, 0)),
            pl.BlockSpec((f, h), lambda i: (0, 0)),
            pl.BlockSpec((f, h), lambda i: (0, 0)),
            pl.BlockSpec((1, h), lambda i: (0, 0)),
            pl.BlockSpec((h, h), lambda i: (0, 0)),
            pl.BlockSpec((1, h), lambda i: (0, 0)),
            pl.BlockSpec((_G, f), lambda i: (0, 0)),
            pl.BlockSpec((_G, h), lambda i: (0, 0)),
            pl.BlockSpec((f, c_out), lambda i: (0, 0)),
            pl.BlockSpec((h, c_out), lambda i: (0, 0)),
            pl.BlockSpec((h, c_out), lambda i: (0, 0)),
            pl.BlockSpec((1, c_out), lambda i: (0, 0)),
        ],
        out_specs=pl.BlockSpec((_G, c_out), lambda i: (0, 0)),
        out_shape=jax.ShapeDtypeStruct((_G, c_out), jnp.float32),
        scratch_shapes=[pltpu.VMEM((_G, h), jnp.float32)],
    )(h1, aggr1, batch2d, w1f1[:f], w1f1[f:], b1f1, w2f1, b2f1,
      pooled0, pooled1, l0_W, l1_W, l2_W, bsum)

    return out


# speedup vs baseline: 8.9227x; 1.1240x over previous
import functools

import jax
import jax.numpy as jnp
import numpy as np
from jax import lax
from jax.experimental import pallas as pl
from jax.experimental.pallas import tpu as pltpu
from jax.experimental.pallas import tpu_sc as plsc

_NC = 2
_NS = 16
_B = 64
_NSL = 5
_GA = 3
_G = 64


def _make_sc_aggr(n_nodes, fc, edges_per_core, edge_stride, src_core_stride):
    assert edges_per_core % _NS == 0
    ew = edges_per_core // _NS
    nfull = ew // _B
    rem = ew - nfull * _B
    n4 = (nfull // _NSL) * _NSL
    assert n4 >= _NSL
    nrow = ((n_nodes + 1 + _NS * 16 - 1) // (_NS * 16)) * (_NS * 16)
    rows_pt = nrow // _NS

    mesh = plsc.VectorSubcoreMesh(core_axis_name="c", subcore_axis_name="s")

    @functools.partial(
        pl.kernel,
        out_type=jax.ShapeDtypeStruct((_NC, nrow, fc), jnp.float32),
        mesh=mesh,
        scratch_types=[
            pltpu.VMEM((_NSL, _B), jnp.int32),
            pltpu.VMEM((_NSL, _B), jnp.int32),
            pltpu.VMEM((_NSL, _B, fc), jnp.float32),
            pltpu.VMEM((16, fc), jnp.float32),
            pltpu.VMEM_SHARED((nrow, fc), jnp.float32),
            [pltpu.SemaphoreType.DMA] * _NSL,
            [pltpu.SemaphoreType.DMA] * _NSL,
            [pltpu.SemaphoreType.DMA] * _NSL,
            [pltpu.SemaphoreType.DMA] * _NSL,
        ],
    )
    def aggr(src_hbm, dst_hbm, table_hbm, out_hbm,
             srcb, dstb, rows, zb, accum, sisem, disem, gsem, ssem):
        c = lax.axis_index("c")
        s = lax.axis_index("s")
        dbase = c * edge_stride + s * ew
        sbase = c * src_core_stride + s * ew

        def sidx_desc(jj, sl):
            return pltpu.make_async_copy(
                src_hbm.at[pl.ds(sbase + jj * _B, _B)], srcb.at[sl], sisem[sl])

        def didx_desc(jj, sl):
            return pltpu.make_async_copy(
                dst_hbm.at[pl.ds(dbase + jj * _B, _B)], dstb.at[sl], disem[sl])

        def gather_desc(sl):
            return pltpu.make_async_copy(
                table_hbm.at[srcb.at[sl]], rows.at[sl], gsem[sl])

        def scatter_start(sl):
            pltpu.async_copy(rows.at[sl], accum.at[dstb.at[sl]], ssem[sl],
                             add=True)

        def scatter_wait(sl):
            pltpu.make_async_copy(rows.at[sl], accum.at[dstb.at[sl]],
                                  ssem[sl]).wait()

        zv = jnp.zeros((16,), jnp.float32)
        for r in range(16):
            for k in range(fc // 16):
                zb[r, pl.ds(k * 16, 16)] = zv
        row0 = s * rows_pt

        @pl.loop(0, rows_pt // 16)
        def _zero(g):
            pltpu.sync_copy(zb, accum.at[pl.ds(row0 + g * 16, 16), :])

        for sl in range(_NSL):
            sidx_desc(sl, sl).start()
            didx_desc(sl, sl).start()
        plsc.subcore_barrier()
        for sl in range(_GA):
            sidx_desc(sl, sl).wait()
            gather_desc(sl).start()

        slag = _NSL - _GA
        @pl.loop(0, n4, step=_NSL)
        def _main(j):
            for sl in range(_NSL):
                jj = j + sl
                sla = (sl + _GA) % _NSL
                gather_desc(sl).wait()
                didx_desc(jj, sl).wait()
                scatter_start(sl)

                @pl.when(jj + _NSL < n4)
                def _pf_sidx():
                    sidx_desc(jj + _NSL, sl).start()

                @pl.when(jnp.logical_and(jj >= slag, jj + _GA < n4))
                def _free_buf():
                    scatter_wait(sla)
                    didx_desc(jj + _GA, sla).start()

                @pl.when(jj + _GA < n4)
                def _next_gather():
                    sidx_desc(jj + _GA, sla).wait()
                    gather_desc(sla).start()

        for sl in range(min(_NSL, n4)):
            scatter_wait(sl)

        for lb in range(n4, nfull):
            sidx_desc(lb, 0).start()
            didx_desc(lb, 0).start()
            sidx_desc(lb, 0).wait()
            didx_desc(lb, 0).wait()
            gather_desc(0).start()
            gather_desc(0).wait()
            scatter_start(0)
            scatter_wait(0)
        if rem:
            zpad = jnp.zeros((16,), jnp.int32)
            dpad = jnp.full((16,), n_nodes, jnp.int32)
            for k in range(_B // 16):
                srcb[0, pl.ds(k * 16, 16)] = zpad
                dstb[0, pl.ds(k * 16, 16)] = dpad
            ra = pltpu.make_async_copy(
                src_hbm.at[pl.ds(sbase + nfull * _B, rem)],
                srcb.at[0, pl.ds(0, rem)], sisem[0])
            rb = pltpu.make_async_copy(
                dst_hbm.at[pl.ds(dbase + nfull * _B, rem)],
                dstb.at[0, pl.ds(0, rem)], disem[0])
            ra.start()
            rb.start()
            ra.wait()
            rb.wait()
            gather_desc(0).start()
            gather_desc(0).wait()
            scatter_start(0)
            scatter_wait(0)

        plsc.subcore_barrier()
        pltpu.sync_copy(accum.at[pl.ds(row0, rows_pt), :],
                        out_hbm.at[c, pl.ds(row0, rows_pt), :])

    return aggr, nrow


def _tc_conv0(n, f, h, rb):
    grid = n // rb

    def body(x_ref, a_ref, b_ref, w1_ref, b1_ref, w2_ref, b2_ref,
             h_ref, p0_ref, p1_ref):
        i = pl.program_id(0)
        x = x_ref[...]
        z = x + a_ref[0] + a_ref[1]
        t = jnp.maximum(
            jnp.dot(z, w1_ref[...], preferred_element_type=jnp.float32)
            + b1_ref[...], 0.0)
        hh = jnp.maximum(
            jnp.dot(t, w2_ref[...], preferred_element_type=jnp.float32)
            + b2_ref[...], 0.0)
        h_ref[0] = hh[:, :f]
        h_ref[1] = hh[:, f:]
        gidx = lax.broadcasted_iota(jnp.int32, (1, _G), 1)
        oh = (b_ref[...] == gidx).astype(jnp.float32)
        dn = (((0,), (0,)), ((), ()))
        p0c = lax.dot_general(oh, x, dn, preferred_element_type=jnp.float32)
        p1c = lax.dot_general(oh, hh, dn, preferred_element_type=jnp.float32)

        @pl.when(i == 0)
        def _init():
            p0_ref[...] = p0c
            p1_ref[...] = p1c

        @pl.when(i > 0)
        def _acc():
            p0_ref[...] += p0c
            p1_ref[...] += p1c

    return body, grid


def _tc_conv1(n, f, h, c_out, rb):
    grid = n // rb

    def body(h1_ref, a_ref, b_ref, w1lo_ref, w1hi_ref, b1_ref, w2_ref, b2_ref,
             p0_ref, p1_ref, l0_ref, l1_ref, l2_ref, bs_ref,
             out_ref, p2_acc):
        i = pl.program_id(0)
        zlo = h1_ref[0] + a_ref[0]
        zhi = h1_ref[1] + a_ref[1]
        t = jnp.maximum(
            jnp.dot(zlo, w1lo_ref[...], preferred_element_type=jnp.float32)
            + jnp.dot(zhi, w1hi_ref[...], preferred_element_type=jnp.float32)
            + b1_ref[...], 0.0)
        h2 = jnp.maximum(
            jnp.dot(t, w2_ref[...], preferred_element_type=jnp.float32)
            + b2_ref[...], 0.0)
        gidx = lax.broadcasted_iota(jnp.int32, (1, _G), 1)
        oh = (b_ref[...] == gidx).astype(jnp.float32)
        dn = (((0,), (0,)), ((), ()))
        p2c = lax.dot_general(oh, h2, dn, preferred_element_type=jnp.float32)

        @pl.when(i == 0)
        def _init():
            p2_acc[...] = p2c

        @pl.when(i > 0)
        def _acc():
            p2_acc[...] += p2c

        @pl.when(i == grid - 1)
        def _final():
            out_ref[...] = (
                jnp.dot(p0_ref[...], l0_ref[...],
                        preferred_element_type=jnp.float32)
                + jnp.dot(p1_ref[...], l1_ref[...],
                          preferred_element_type=jnp.float32)
                + jnp.dot(p2_acc[...], l2_ref[...],
                          preferred_element_type=jnp.float32)
                + bs_ref[...])

    return body, grid


def kernel(x, edge_index, batch,
           c0_W1, c0_b1, c0_g1, c0_be1, c0_W2, c0_b2, bn0_g, bn0_b,
           c1_W1, c1_b1, c1_g1, c1_be1, c1_W2, c1_b2, bn1_g, bn1_b,
           l0_W, l0_b, l1_W, l1_b, l2_W, l2_b):
    n, f = x.shape
    e = edge_index.shape[1]
    h = c0_W2.shape[0]
    c_out = l0_W.shape[1]
    rb = 2000

    src = edge_index[0]
    dst = edge_index[1]
    batch2d = batch.reshape(n, 1)

    inv = float(1.0 / np.sqrt(1.0 + 1e-5))
    s1a = c0_g1 * inv
    w1f0 = c0_W1 * s1a[None, :]
    b1f0 = (c0_b1 * s1a + c0_be1).reshape(1, h)
    s0 = bn0_g * inv
    w2f0 = c0_W2 * s0[None, :]
    b2f0 = (c0_b2 * s0 + bn0_b).reshape(1, h)
    s1b = c1_g1 * inv
    w1f1 = c1_W1 * s1b[None, :]
    b1f1 = (c1_b1 * s1b + c1_be1).reshape(1, h)
    s1 = bn1_g * inv
    w2f1 = c1_W2 * s1[None, :]
    b2f1 = (c1_b2 * s1 + bn1_b).reshape(1, h)
    bsum = (l0_b + l1_b + l2_b).reshape(1, c_out)

    sc0, nrow = _make_sc_aggr(n, f, e // 2, e // 2, e // 2)
    aggr0 = sc0(src, dst, x)

    body0, grid0 = _tc_conv0(n, f, h, rb)
    h1, pooled0, pooled1 = pl.pallas_call(
        body0,
        grid=(grid0,),
        in_specs=[
            pl.BlockSpec((rb, f), lambda i: (i, 0)),
            pl.BlockSpec((2, rb, f), lambda i: (0, i, 0)),
            pl.BlockSpec((rb, 1), lambda i: (i, 0)),
            pl.BlockSpec((f, h), lambda i: (0, 0)),
            pl.BlockSpec((1, h), lambda i: (0, 0)),
            pl.BlockSpec((h, h), lambda i: (0, 0)),
            pl.BlockSpec((1, h), lambda i: (0, 0)),
        ],
        out_specs=[
            pl.BlockSpec((2, rb, f), lambda i: (0, i, 0)),
            pl.BlockSpec((_G, f), lambda i: (0, 0)),
            pl.BlockSpec((_G, h), lambda i: (0, 0)),
        ],
        out_shape=[
            jax.ShapeDtypeStruct((2, n, f), jnp.float32),
            jax.ShapeDtypeStruct((_G, f), jnp.float32),
            jax.ShapeDtypeStruct((_G, h), jnp.float32),
        ],
    )(x, aggr0, batch2d, w1f0, b1f0, w2f0, b2f0)

    h1flat = h1.reshape(2 * n, f)
    src_cat = jnp.concatenate([src, src + n])
    sc1, nrow1 = _make_sc_aggr(n, f, e, 0, e)
    aggr1 = sc1(src_cat, dst, h1flat)

    body1, grid1 = _tc_conv1(n, f, h, c_out, rb)
    out = pl.pallas_call(
        body1,
        grid=(grid1,),
        in_specs=[
            pl.BlockSpec((2, rb, f), lambda i: (0, i, 0)),
            pl.BlockSpec((2, rb, f), lambda i: (0, i, 0)),
            pl.BlockSpec((rb, 1), lambda i: (i, 0)),
            pl.BlockSpec((f, h), lambda i: (0, 0)),
            pl.BlockSpec((f, h), lambda i: (0, 0)),
            pl.BlockSpec((1, h), lambda i: (0, 0)),
            pl.BlockSpec((h, h), lambda i: (0, 0)),
            pl.BlockSpec((1, h), lambda i: (0, 0)),
            pl.BlockSpec((_G, f), lambda i: (0, 0)),
            pl.BlockSpec((_G, h), lambda i: (0, 0)),
            pl.BlockSpec((f, c_out), lambda i: (0, 0)),
            pl.BlockSpec((h, c_out), lambda i: (0, 0)),
            pl.BlockSpec((h, c_out), lambda i: (0, 0)),
            pl.BlockSpec((1, c_out), lambda i: (0, 0)),
        ],
        out_specs=pl.BlockSpec((_G, c_out), lambda i: (0, 0)),
        out_shape=jax.ShapeDtypeStruct((_G, c_out), jnp.float32),
        scratch_shapes=[pltpu.VMEM((_G, h), jnp.float32)],
    )(h1, aggr1, batch2d, w1f1[:f], w1f1[f:], b1f1, w2f1, b2f1,
      pooled0, pooled1, l0_W, l1_W, l2_W, bsum)

    return out
